# baseline (speedup 1.0000x reference)
import functools

import jax
import jax.numpy as jnp
from jax import lax
from jax.experimental import pallas as pl
from jax.experimental.pallas import tpu as pltpu
from jax.experimental.pallas import tpu_sc as plsc

DIM = 64
LANES = 16
NSL = DIM // LANES
NBUF = 5
LEAD = 2


def kernel(x, tok_table, pos_table):
    B, S = x.shape
    NW = 32
    BW = B // NW
    ST = S // 8
    BT = B // 128
    DT = DIM // 8

    xview = x.astype(jnp.int32).T.reshape(ST, 8, BT, 128).transpose(0, 2, 1, 3)
    pos = pos_table[:S]

    mesh = plsc.VectorSubcoreMesh(core_axis_name="c", subcore_axis_name="s")

    @functools.partial(
        pl.kernel,
        mesh=mesh,
        out_type=jax.ShapeDtypeStruct((S, DT, BT, 8, 128), jnp.float32),
        compiler_params=pltpu.CompilerParams(
            use_tc_tiling_on_sc=False, needs_layout_passes=False),
        scratch_types=[
            pltpu.VMEM((ST, 8, 128), jnp.int32),
            pltpu.VMEM((NBUF, BW, DIM), jnp.float32),
            pltpu.VMEM((NBUF, DT, 8, 129), jnp.float32),
            pltpu.VMEM((S, DIM), jnp.float32),
            [pltpu.SemaphoreType.DMA for _ in range(NBUF)],
            [pltpu.SemaphoreType.DMA for _ in range(NBUF)],
        ],
    )
    def sc_kernel(x_hbm, tok_hbm, pos_hbm, out_hbm,
                  idx_v, rows_b, plane_b, pos_v, gsems, ssems):
        wid = lax.axis_index("s") * 2 + lax.axis_index("c")
        pltpu.sync_copy(pos_hbm.at[pl.ds(0, S)], pos_v)
        pltpu.sync_copy(x_hbm.at[pl.ds(0, ST), wid], idx_v)

        iota = lax.iota(jnp.int32, LANES)
        dt_sel = [(iota + k * LANES) >> 3 for k in range(NSL)]
        d8_sel = [(iota + k * LANES) & 7 for k in range(NSL)]

        def gather_cp(p, b):
            return pltpu.make_async_copy(
                tok_hbm.at[idx_v.at[p // 8, p % 8]], rows_b.at[b], gsems[b])

        def scatter_cp(p, b):
            return pltpu.make_async_copy(
                plane_b.at[b, :, :, pl.ds(0, 128)],
                out_hbm.at[p, pl.ds(0, DT), wid], ssems[b])

        for q in range(LEAD):
            gather_cp(q, q).start()

        def outer(pp, carry):
            for u in range(NBUF):
                p = pp * NBUF + u
                b2 = (u + LEAD) % NBUF

                @pl.when(p >= NBUF - LEAD)
                def _():
                    scatter_cp(p - (NBUF - LEAD), b2).wait()

                @pl.when(p + LEAD < S)
                def _():
                    gather_cp(p + LEAD, b2).start()

                gather_cp(p, u).wait()
                pregs = [pos_v[p, pl.ds(k * LANES, LANES)] for k in range(NSL)]
                rows = rows_b.at[u]
                plane = plane_b.at[u]

                @plsc.parallel_loop(0, BW, unroll=4)
                def _(r):
                    bspl = jnp.full((LANES,), r, jnp.int32)
                    for k in range(NSL):
                        v = rows[r, pl.ds(k * LANES, LANES)] + pregs[k]
                        plsc.store_scatter(
                            plane, [dt_sel[k], d8_sel[k], bspl], v)

                scatter_cp(p, u).start()
            return carry

        lax.fori_loop(0, S // NBUF, outer, 0)
        for q in range(NBUF - LEAD):
            pq = S - (NBUF - LEAD) + q
            scatter_cp(pq, pq % NBUF).wait()

    out5 = sc_kernel(xview, tok_table, pos)
    return out5.transpose(2, 4, 0, 1, 3).reshape(B, S, DIM)


# speedup vs baseline: 1.0061x; 1.0061x over previous
import functools

import jax
import jax.numpy as jnp
from jax import lax
from jax.experimental import pallas as pl
from jax.experimental.pallas import tpu as pltpu
from jax.experimental.pallas import tpu_sc as plsc

DIM = 64
LANES = 16
NSL = DIM // LANES
NBUF = 5
LEAD = 3


def kernel(x, tok_table, pos_table):
    B, S = x.shape
    NW = 32
    BW = B // NW
    ST = S // 8
    BT = B // 128
    DT = DIM // 8

    xview = x.astype(jnp.int32).T.reshape(ST, 8, BT, 128).transpose(0, 2, 1, 3)
    pos = pos_table[:S]

    mesh = plsc.VectorSubcoreMesh(core_axis_name="c", subcore_axis_name="s")

    @functools.partial(
        pl.kernel,
        mesh=mesh,
        out_type=jax.ShapeDtypeStruct((S, DT, BT, 8, 128), jnp.float32),
        compiler_params=pltpu.CompilerParams(
            use_tc_tiling_on_sc=False, needs_layout_passes=False,
            disable_bounds_checks=True),
        scratch_types=[
            pltpu.VMEM((ST, 8, 128), jnp.int32),
            pltpu.VMEM((NBUF, BW, DIM), jnp.float32),
            pltpu.VMEM((NBUF, DT, 8, 129), jnp.float32),
            pltpu.VMEM((S, DIM), jnp.float32),
            [pltpu.SemaphoreType.DMA for _ in range(NBUF)],
            [pltpu.SemaphoreType.DMA for _ in range(NBUF)],
        ],
    )
    def sc_kernel(x_hbm, tok_hbm, pos_hbm, out_hbm,
                  idx_v, rows_b, plane_b, pos_v, gsems, ssems):
        wid = lax.axis_index("s") * 2 + lax.axis_index("c")
        pltpu.sync_copy(pos_hbm.at[pl.ds(0, S)], pos_v)
        pltpu.sync_copy(x_hbm.at[pl.ds(0, ST), wid], idx_v)

        iota = lax.iota(jnp.int32, LANES)
        dt_sel = [(iota + k * LANES) >> 3 for k in range(NSL)]
        d8_sel = [(iota + k * LANES) & 7 for k in range(NSL)]

        def gather_cp(p, b):
            return pltpu.make_async_copy(
                tok_hbm.at[idx_v.at[p // 8, p % 8]], rows_b.at[b], gsems[b])

        def scatter_cp(p, b):
            return pltpu.make_async_copy(
                plane_b.at[b, :, :, pl.ds(0, 128)],
                out_hbm.at[p, pl.ds(0, DT), wid], ssems[b])

        for q in range(LEAD):
            gather_cp(q, q).start()

        def outer(pp, carry):
            for u in range(NBUF):
                p = pp * NBUF + u
                b2 = (u + LEAD) % NBUF

                @pl.when(p >= NBUF - LEAD)
                def _():
                    scatter_cp(p - (NBUF - LEAD), b2).wait()

                @pl.when(p + LEAD < S)
                def _():
                    gather_cp(p + LEAD, b2).start()

                gather_cp(p, u).wait()
                pregs = [pos_v[p, pl.ds(k * LANES, LANES)] for k in range(NSL)]
                rows = rows_b.at[u]
                plane = plane_b.at[u]

                @plsc.parallel_loop(0, BW, unroll=4)
                def _(r):
                    bspl = jnp.full((LANES,), r, jnp.int32)
                    for k in range(NSL):
                        v = rows[r, pl.ds(k * LANES, LANES)] + pregs[k]
                        plsc.store_scatter(
                            plane, [dt_sel[k], d8_sel[k], bspl], v)

                scatter_cp(p, u).start()
            return carry

        lax.fori_loop(0, S // NBUF, outer, 0)
        for q in range(NBUF - LEAD):
            pq = S - (NBUF - LEAD) + q
            scatter_cp(pq, pq % NBUF).wait()

    out5 = sc_kernel(xview, tok_table, pos)
    return out5.transpose(2, 4, 0, 1, 3).reshape(B, S, DIM)
